# baseline (speedup 1.0000x reference)
import functools

import jax
import jax.numpy as jnp
from jax import lax
from jax.experimental import pallas as pl
from jax.experimental.pallas import tpu as pltpu
from jax.experimental.pallas import tpu_sc as plsc

B, L, D = 8, 4096, 512
LP = 32
LQ = 128
NT = 6.0
ROWS_PER_SC = 4
QUARTERS = 4
JSEG = L // QUARTERS
DUMP = L + 15
CH = 64
NCH = JSEG // CH


def _tc_body(lens_ref, tokens_ref, wq_ref, wk_ref, skt_ref,
             probs_ref, bmask_ref, aux_ref):
    b = pl.program_id(0)
    x = tokens_ref[0]
    q = lax.dot_general(x, wq_ref[...], (((1,), (1,)), ((), ())),
                        preferred_element_type=jnp.float32)
    k = lax.dot_general(x, wk_ref[...], (((1,), (1,)), ((), ())),
                        preferred_element_type=jnp.float32)
    ksim = jnp.concatenate([skt_ref[...], k[:-1, :]], axis=0)
    dot = jnp.sum(q * ksim, axis=1)
    qn = jnp.sqrt(jnp.sum(q * q, axis=1))
    kn = jnp.sqrt(jnp.sum(ksim * ksim, axis=1))
    cos = dot / jnp.maximum(qn * kn, 1e-8)
    probs = ((1.0 - cos) * 0.5).reshape(LP, LQ)

    pos = (lax.broadcasted_iota(jnp.int32, (LP, LQ), 0) * LQ
           + lax.broadcasted_iota(jnp.int32, (LP, LQ), 1))
    len_b = lens_ref[b]
    seq = pos < len_b
    bmask = ((probs > 0.5) | (pos == 0)) & seq

    probs_ref[0] = probs
    bmask_ref[0] = bmask.astype(jnp.int32)

    len_f = len_b.astype(jnp.float32)
    bm_f = bmask.astype(jnp.float32)
    nc_f = jnp.sum(bm_f)
    g_row = jnp.sum(jnp.where(seq, probs, 0.0)) / len_f
    f_row = nc_f / len_f
    aux = NT / (NT - 1.0) * ((NT - 1.0) * f_row * g_row
                             + (1.0 - f_row) * (1.0 - g_row))
    aux_ref[0] = jnp.full((1, LQ), aux, dtype=jnp.float32)


def _tc_probs(tokens, lens, w_q, w_k, skt):
    return pl.pallas_call(
        _tc_body,
        grid=(B,),
        in_specs=[
            pl.BlockSpec(memory_space=pltpu.SMEM),
            pl.BlockSpec((1, L, D), lambda b: (b, 0, 0)),
            pl.BlockSpec((D, D), lambda b: (0, 0)),
            pl.BlockSpec((D, D), lambda b: (0, 0)),
            pl.BlockSpec((1, D), lambda b: (0, 0)),
        ],
        out_specs=[
            pl.BlockSpec((1, LP, LQ), lambda b: (b, 0, 0)),
            pl.BlockSpec((1, LP, LQ), lambda b: (b, 0, 0)),
            pl.BlockSpec((1, 1, LQ), lambda b: (b, 0, 0)),
        ],
        out_shape=[
            jax.ShapeDtypeStruct((B, LP, LQ), jnp.float32),
            jax.ShapeDtypeStruct((B, LP, LQ), jnp.int32),
            jax.ShapeDtypeStruct((B, 1, LQ), jnp.float32),
        ],
        compiler_params=pltpu.CompilerParams(
            dimension_semantics=("arbitrary",)),
    )(lens, tokens, w_q, w_k, skt)


def _sc_body(tokens_ref, probs_ref, bmask_ref, lens_ref,
             ds_ref, gates_ref, cl_ref,
             lens_v, bm_v, pr_v, sel_v, bp_v, cl_v, gt_v,
             rows_v, rows2_v, zeros_v, sem, sem2, wsem, wsem2):
    c = lax.axis_index("c")
    s = lax.axis_index("s")
    wid = c * 16 + s
    row = wid // QUARTERS
    quarter = wid % QUARTERS
    jbase = quarter * JSEG
    base = row * L

    pltpu.sync_copy(lens_ref, lens_v)
    pltpu.sync_copy(bmask_ref.at[row], bm_v)
    pltpu.sync_copy(probs_ref.at[row], pr_v)
    len_r = lens_v[pl.ds(row, 16)][0]

    def _init(g, _):
        z16 = jnp.zeros((16,), jnp.int32)
        sel_v[pl.ds(g * 16, 16)] = z16
        bp_v[pl.ds(g * 16, 16)] = z16.astype(jnp.float32)
        return 0
    lax.fori_loop(0, (L + 16) // 16, _init, 0)

    def _compact(g, off):
        bm = bm_v[pl.ds(g * 16, 16)]
        msk = bm > 0
        posv = g * 16 + lax.iota(jnp.int32, 16)
        csum = plsc.cumsum(bm)
        dest = jnp.where(msk, off + csum - bm, DUMP)
        plsc.store_scatter(sel_v, [dest], base + posv)
        plsc.store_scatter(bp_v, [dest], pr_v[pl.ds(g * 16, 16)])
        return off + csum[15]
    nc = lax.fori_loop(0, L // 16, _compact, 0)
    lane0 = lax.iota(jnp.int32, 16) == 0
    plsc.store_scatter(sel_v, [jnp.where(lane0, nc, DUMP)],
                       jnp.full((16,), base + len_r, jnp.int32))

    @pl.when(quarter == 0)
    def _derive_out():
        def _derive(g, _):
            a = sel_v[pl.ds(g * 16, 16)]
            b2 = sel_v[pl.ds(g * 16 + 1, 16)]
            jv = g * 16 + lax.iota(jnp.int32, 16)
            cl_v[pl.ds(g * 16, 16)] = jnp.where(jv < nc, b2 - a, 0)
            gt_v[pl.ds(g * 16, 16)] = 1.0 - bp_v[pl.ds(g * 16, 16)]
            return 0
        lax.fori_loop(0, L // 16, _derive, 0)
        pltpu.sync_copy(cl_v, cl_ref.at[row])
        pltpu.sync_copy(gt_v, gates_ref.at[row])

    def _zinit(j, _):
        for dd in range(D // 16):
            zeros_v[j, pl.ds(dd * 16, 16)] = jnp.zeros((16,), jnp.float32)
        return 0
    lax.fori_loop(0, CH, _zinit, 0)

    def _j0(k):
        return (quarter + QUARTERS * k) * CH

    def _active(k):
        return jnp.logical_and(k < NCH, _j0(k) < nc)

    def _issue(k, buf, gsem):
        @pl.when(_active(k))
        def _():
            pltpu.async_copy(
                tokens_ref.at[sel_v.at[pl.ds(_j0(k), CH)]], buf, gsem)

    def _drain(k, buf, gsem, wsem):
        j0 = _j0(k)
        out_slice = ds_ref.at[pl.ds(base + j0, CH)]

        @pl.when(j0 < nc)
        def _():
            pltpu.make_async_copy(
                tokens_ref.at[sel_v.at[pl.ds(j0, CH)]], buf, gsem).wait()

            def _scale(j, _2):
                scl = bp_v[pl.ds(j0 + j, 16)][0]
                for dd in range(D // 16):
                    v = buf[j, pl.ds(dd * 16, 16)]
                    buf[j, pl.ds(dd * 16, 16)] = v * scl
                return 0
            lax.fori_loop(0, CH, _scale, 0)
            pltpu.async_copy(buf, out_slice, wsem)

        @pl.when(jnp.logical_not(j0 < nc))
        def _():
            pltpu.async_copy(zeros_v, out_slice, wsem)

    def _waitw(k, buf, wsem):
        pltpu.make_async_copy(buf, ds_ref.at[pl.ds(base + _j0(k), CH)],
                              wsem).wait()

    _issue(0, rows_v, sem)
    _issue(1, rows2_v, sem2)

    def _pair(i, _):
        g0 = 2 * i
        _drain(g0, rows_v, sem, wsem)
        _drain(g0 + 1, rows2_v, sem2, wsem2)
        _waitw(g0, rows_v, wsem)
        _issue(g0 + 2, rows_v, sem)
        _waitw(g0 + 1, rows2_v, wsem2)
        _issue(g0 + 3, rows2_v, sem2)
        return 0
    lax.fori_loop(0, NCH // 2, _pair, 0)


@functools.partial(
    pl.kernel,
    out_type=[
        jax.ShapeDtypeStruct((B * L, D), jnp.float32),
        jax.ShapeDtypeStruct((B, L), jnp.float32),
        jax.ShapeDtypeStruct((B, L), jnp.int32),
    ],
    mesh=plsc.VectorSubcoreMesh(core_axis_name="c", subcore_axis_name="s"),
    compiler_params=pltpu.CompilerParams(needs_layout_passes=False),
    scratch_types=[
        pltpu.VMEM((32,), jnp.int32),
        pltpu.VMEM((L,), jnp.int32),
        pltpu.VMEM((L,), jnp.float32),
        pltpu.VMEM((L + 16,), jnp.int32),
        pltpu.VMEM((L + 16,), jnp.float32),
        pltpu.VMEM((L,), jnp.int32),
        pltpu.VMEM((L,), jnp.float32),
        pltpu.VMEM((CH, D), jnp.float32),
        pltpu.VMEM((CH, D), jnp.float32),
        pltpu.VMEM((CH, D), jnp.float32),
        pltpu.SemaphoreType.DMA,
        pltpu.SemaphoreType.DMA,
        pltpu.SemaphoreType.DMA,
        pltpu.SemaphoreType.DMA,
    ],
)
def _sc_compact(tokens_ref, probs_ref, bmask_ref, lens_ref,
                ds_ref, gates_ref, cl_ref, *scratch):
    _sc_body(tokens_ref, probs_ref, bmask_ref, lens_ref,
             ds_ref, gates_ref, cl_ref, *scratch)


def kernel(tokens, lens, W_qk, start_key_token):
    w_q = W_qk[:D]
    w_k = W_qk[D:]
    skt = start_key_token.reshape(1, D)
    probs3, bmask3, aux3 = _tc_probs(tokens, lens, w_q, w_k, skt)
    probs = probs3.reshape(B, L)
    bmask_i = bmask3.reshape(B, L)
    lens_pad = jnp.concatenate([lens, jnp.zeros((32 - B,), jnp.int32)])

    ds2, gates, chunk_lens = _sc_compact(
        tokens.reshape(B * L, D), probs, bmask_i, lens_pad)

    downsampled = ds2.reshape(B, L, D)
    boundary_mask = bmask_i.astype(bool)
    upsampler_output_scale = jnp.ones((B, L), jnp.float32)
    weighted_aux_loss = jnp.mean(aux3[:, 0, 0]) * 0.03
    return (downsampled, gates, chunk_lens, boundary_mask,
            upsampler_output_scale, weighted_aux_loss)


# speedup vs baseline: 1.0349x; 1.0349x over previous
import functools

import jax
import jax.numpy as jnp
from jax import lax
from jax.experimental import pallas as pl
from jax.experimental.pallas import tpu as pltpu
from jax.experimental.pallas import tpu_sc as plsc

B, L, D = 8, 4096, 512
LP = 32
LQ = 128
NT = 6.0
ROWS_PER_SC = 4
QUARTERS = 4
JSEG = L // QUARTERS
DUMP = L + 15
CH = 64
NCH = JSEG // CH


NBL = 4
T = L // NBL
TP = T // LQ


def _tc_body(lens_ref, tokens_ref, wq_ref, wk_ref, skt_ref,
             probs_ref, bmask_ref, aux_ref, kcarry, acc):
    b = pl.program_id(0)
    i = pl.program_id(1)
    len_b = lens_ref[b]

    @pl.when(i * T < len_b)
    def _active():
        x = tokens_ref[0]
        q = lax.dot_general(x, wq_ref[...], (((1,), (1,)), ((), ())),
                            preferred_element_type=jnp.float32)
        k = lax.dot_general(x, wk_ref[...], (((1,), (1,)), ((), ())),
                            preferred_element_type=jnp.float32)
        prev = jnp.where(i == 0, skt_ref[...], kcarry[...])
        ksim = jnp.concatenate([prev, k[:-1, :]], axis=0)
        kcarry[...] = k[-1:, :]
        dot = jnp.sum(q * ksim, axis=1)
        qn = jnp.sqrt(jnp.sum(q * q, axis=1))
        kn = jnp.sqrt(jnp.sum(ksim * ksim, axis=1))
        cos = dot / jnp.maximum(qn * kn, 1e-8)
        probs = ((1.0 - cos) * 0.5).reshape(TP, LQ)

        pos = (i * T + lax.broadcasted_iota(jnp.int32, (TP, LQ), 0) * LQ
               + lax.broadcasted_iota(jnp.int32, (TP, LQ), 1))
        seq = pos < len_b
        bmask = ((probs > 0.5) | (pos == 0)) & seq

        probs_ref[0] = probs
        bmask_ref[0] = bmask.astype(jnp.int32)

        nc_part = jnp.sum(bmask.astype(jnp.float32))
        g_part = jnp.sum(jnp.where(seq, probs, 0.0))
        nc_f = jnp.where(i == 0, nc_part, acc[0] + nc_part)
        g_sum = jnp.where(i == 0, g_part, acc[1] + g_part)
        acc[0] = nc_f
        acc[1] = g_sum

        len_f = len_b.astype(jnp.float32)
        g_row = g_sum / len_f
        f_row = nc_f / len_f
        aux = NT / (NT - 1.0) * ((NT - 1.0) * f_row * g_row
                                 + (1.0 - f_row) * (1.0 - g_row))
        aux_ref[0] = jnp.full((1, LQ), aux, dtype=jnp.float32)

    @pl.when(jnp.logical_not(i * T < len_b))
    def _masked_tail():
        probs_ref[0] = jnp.zeros((TP, LQ), jnp.float32)
        bmask_ref[0] = jnp.zeros((TP, LQ), jnp.int32)


def _tc_probs(tokens, lens, w_q, w_k, skt):
    return pl.pallas_call(
        _tc_body,
        grid=(B, NBL),
        in_specs=[
            pl.BlockSpec(memory_space=pltpu.SMEM),
            pl.BlockSpec((1, T, D), lambda b, i: (b, i, 0)),
            pl.BlockSpec((D, D), lambda b, i: (0, 0)),
            pl.BlockSpec((D, D), lambda b, i: (0, 0)),
            pl.BlockSpec((1, D), lambda b, i: (0, 0)),
        ],
        out_specs=[
            pl.BlockSpec((1, TP, LQ), lambda b, i: (b, i, 0)),
            pl.BlockSpec((1, TP, LQ), lambda b, i: (b, i, 0)),
            pl.BlockSpec((1, 1, LQ), lambda b, i: (b, 0, 0)),
        ],
        out_shape=[
            jax.ShapeDtypeStruct((B, LP, LQ), jnp.float32),
            jax.ShapeDtypeStruct((B, LP, LQ), jnp.int32),
            jax.ShapeDtypeStruct((B, 1, LQ), jnp.float32),
        ],
        scratch_shapes=[
            pltpu.VMEM((1, D), jnp.float32),
            pltpu.SMEM((2,), jnp.float32),
        ],
        compiler_params=pltpu.CompilerParams(
            dimension_semantics=("arbitrary", "arbitrary")),
    )(lens, tokens, w_q, w_k, skt)


def _sc_body(tokens_ref, probs_ref, bmask_ref, lens_ref,
             ds_ref, gates_ref, cl_ref,
             lens_v, bm_v, pr_v, sel_v, bp_v, cl_v, gt_v,
             rows_v, rows2_v, zeros_v, sem, sem2, wsem, wsem2):
    c = lax.axis_index("c")
    s = lax.axis_index("s")
    wid = c * 16 + s
    row = wid // QUARTERS
    quarter = wid % QUARTERS
    jbase = quarter * JSEG
    base = row * L

    pltpu.sync_copy(lens_ref, lens_v)
    pltpu.sync_copy(bmask_ref.at[row], bm_v)
    pltpu.sync_copy(probs_ref.at[row], pr_v)
    len_r = lens_v[pl.ds(row, 16)][0]

    def _init(g, _):
        z16 = jnp.zeros((16,), jnp.int32)
        sel_v[pl.ds(g * 16, 16)] = z16
        bp_v[pl.ds(g * 16, 16)] = z16.astype(jnp.float32)
        return 0
    lax.fori_loop(0, (L + 16) // 16, _init, 0)

    def _compact(g, off):
        bm = bm_v[pl.ds(g * 16, 16)]
        msk = bm > 0
        posv = g * 16 + lax.iota(jnp.int32, 16)
        csum = plsc.cumsum(bm)
        dest = jnp.where(msk, off + csum - bm, DUMP)
        plsc.store_scatter(sel_v, [dest], base + posv)
        plsc.store_scatter(bp_v, [dest], pr_v[pl.ds(g * 16, 16)])
        return off + csum[15]
    nc = lax.fori_loop(0, L // 16, _compact, 0)
    lane0 = lax.iota(jnp.int32, 16) == 0
    plsc.store_scatter(sel_v, [jnp.where(lane0, nc, DUMP)],
                       jnp.full((16,), base + len_r, jnp.int32))

    @pl.when(quarter == 0)
    def _derive_out():
        def _derive(g, _):
            a = sel_v[pl.ds(g * 16, 16)]
            b2 = sel_v[pl.ds(g * 16 + 1, 16)]
            jv = g * 16 + lax.iota(jnp.int32, 16)
            cl_v[pl.ds(g * 16, 16)] = jnp.where(jv < nc, b2 - a, 0)
            gt_v[pl.ds(g * 16, 16)] = 1.0 - bp_v[pl.ds(g * 16, 16)]
            return 0
        lax.fori_loop(0, L // 16, _derive, 0)
        pltpu.sync_copy(cl_v, cl_ref.at[row])
        pltpu.sync_copy(gt_v, gates_ref.at[row])

    def _zinit(j, _):
        for dd in range(D // 16):
            zeros_v[j, pl.ds(dd * 16, 16)] = jnp.zeros((16,), jnp.float32)
        return 0
    lax.fori_loop(0, CH, _zinit, 0)

    def _j0(k):
        return (quarter + QUARTERS * k) * CH

    def _active(k):
        return jnp.logical_and(k < NCH, _j0(k) < nc)

    def _issue(k, buf, gsem):
        @pl.when(_active(k))
        def _():
            pltpu.async_copy(
                tokens_ref.at[sel_v.at[pl.ds(_j0(k), CH)]], buf, gsem)

    def _drain(k, buf, gsem, wsem):
        j0 = _j0(k)
        out_slice = ds_ref.at[pl.ds(base + j0, CH)]

        @pl.when(j0 < nc)
        def _():
            pltpu.make_async_copy(
                tokens_ref.at[sel_v.at[pl.ds(j0, CH)]], buf, gsem).wait()

            def _scale(j, _2):
                scl = bp_v[pl.ds(j0 + j, 16)][0]
                for dd in range(D // 16):
                    v = buf[j, pl.ds(dd * 16, 16)]
                    buf[j, pl.ds(dd * 16, 16)] = v * scl
                return 0
            lax.fori_loop(0, CH, _scale, 0)
            pltpu.async_copy(buf, out_slice, wsem)

        @pl.when(jnp.logical_not(j0 < nc))
        def _():
            pltpu.async_copy(zeros_v, out_slice, wsem)

    def _waitw(k, buf, wsem):
        pltpu.make_async_copy(buf, ds_ref.at[pl.ds(base + _j0(k), CH)],
                              wsem).wait()

    _issue(0, rows_v, sem)
    _issue(1, rows2_v, sem2)

    def _pair(i, _):
        g0 = 2 * i
        _drain(g0, rows_v, sem, wsem)
        _drain(g0 + 1, rows2_v, sem2, wsem2)
        _waitw(g0, rows_v, wsem)
        _issue(g0 + 2, rows_v, sem)
        _waitw(g0 + 1, rows2_v, wsem2)
        _issue(g0 + 3, rows2_v, sem2)
        return 0
    lax.fori_loop(0, NCH // 2, _pair, 0)


@functools.partial(
    pl.kernel,
    out_type=[
        jax.ShapeDtypeStruct((B * L, D), jnp.float32),
        jax.ShapeDtypeStruct((B, L), jnp.float32),
        jax.ShapeDtypeStruct((B, L), jnp.int32),
    ],
    mesh=plsc.VectorSubcoreMesh(core_axis_name="c", subcore_axis_name="s"),
    compiler_params=pltpu.CompilerParams(needs_layout_passes=False),
    scratch_types=[
        pltpu.VMEM((32,), jnp.int32),
        pltpu.VMEM((L,), jnp.int32),
        pltpu.VMEM((L,), jnp.float32),
        pltpu.VMEM((L + 16,), jnp.int32),
        pltpu.VMEM((L + 16,), jnp.float32),
        pltpu.VMEM((L,), jnp.int32),
        pltpu.VMEM((L,), jnp.float32),
        pltpu.VMEM((CH, D), jnp.float32),
        pltpu.VMEM((CH, D), jnp.float32),
        pltpu.VMEM((CH, D), jnp.float32),
        pltpu.SemaphoreType.DMA,
        pltpu.SemaphoreType.DMA,
        pltpu.SemaphoreType.DMA,
        pltpu.SemaphoreType.DMA,
    ],
)
def _sc_compact(tokens_ref, probs_ref, bmask_ref, lens_ref,
                ds_ref, gates_ref, cl_ref, *scratch):
    _sc_body(tokens_ref, probs_ref, bmask_ref, lens_ref,
             ds_ref, gates_ref, cl_ref, *scratch)


def kernel(tokens, lens, W_qk, start_key_token):
    w_q = W_qk[:D]
    w_k = W_qk[D:]
    skt = start_key_token.reshape(1, D)
    probs3, bmask3, aux3 = _tc_probs(tokens, lens, w_q, w_k, skt)
    probs = probs3.reshape(B, L)
    bmask_i = bmask3.reshape(B, L)
    lens_pad = jnp.concatenate([lens, jnp.zeros((32 - B,), jnp.int32)])

    ds2, gates, chunk_lens = _sc_compact(
        tokens.reshape(B * L, D), probs, bmask_i, lens_pad)

    downsampled = ds2.reshape(B, L, D)
    boundary_mask = bmask_i.astype(bool)
    upsampler_output_scale = jnp.ones((B, L), jnp.float32)
    weighted_aux_loss = jnp.mean(aux3[:, 0, 0]) * 0.03
    return (downsampled, gates, chunk_lens, boundary_mask,
            upsampler_output_scale, weighted_aux_loss)


# speedup vs baseline: 1.0370x; 1.0021x over previous
import functools

import jax
import jax.numpy as jnp
from jax import lax
from jax.experimental import pallas as pl
from jax.experimental.pallas import tpu as pltpu
from jax.experimental.pallas import tpu_sc as plsc

B, L, D = 8, 4096, 512
LP = 32
LQ = 128
NT = 6.0
ROWS_PER_SC = 4
QUARTERS = 4
JSEG = L // QUARTERS
DUMP = L + 15
CH = 64
NCH = JSEG // CH


NBL = 4
T = L // NBL
TP = T // LQ


def _tc_body(lens_ref, tokens_ref, wq_ref, wk_ref, skt_ref,
             probs_ref, bmask_ref, aux_ref, kcarry, acc):
    b = pl.program_id(0)
    i = pl.program_id(1)
    len_b = lens_ref[b]

    @pl.when(i * T < len_b)
    def _active():
        x = tokens_ref[0]
        q = lax.dot_general(x, wq_ref[...], (((1,), (1,)), ((), ())),
                            preferred_element_type=jnp.float32)
        k = lax.dot_general(x, wk_ref[...], (((1,), (1,)), ((), ())),
                            preferred_element_type=jnp.float32)
        prev = jnp.where(i == 0, skt_ref[...], kcarry[...])
        ksim = jnp.concatenate([prev, k[:-1, :]], axis=0)
        kcarry[...] = k[-1:, :]
        dot = jnp.sum(q * ksim, axis=1)
        qn = jnp.sqrt(jnp.sum(q * q, axis=1))
        kn = jnp.sqrt(jnp.sum(ksim * ksim, axis=1))
        cos = dot / jnp.maximum(qn * kn, 1e-8)
        probs = ((1.0 - cos) * 0.5).reshape(TP, LQ)

        pos = (i * T + lax.broadcasted_iota(jnp.int32, (TP, LQ), 0) * LQ
               + lax.broadcasted_iota(jnp.int32, (TP, LQ), 1))
        seq = pos < len_b
        bmask = ((probs > 0.5) | (pos == 0)) & seq

        probs_ref[0] = probs
        bmask_ref[0] = bmask.astype(jnp.int32)

        nc_part = jnp.sum(bmask.astype(jnp.float32))
        g_part = jnp.sum(jnp.where(seq, probs, 0.0))
        nc_f = jnp.where(i == 0, nc_part, acc[0] + nc_part)
        g_sum = jnp.where(i == 0, g_part, acc[1] + g_part)
        acc[0] = nc_f
        acc[1] = g_sum

        len_f = len_b.astype(jnp.float32)
        g_row = g_sum / len_f
        f_row = nc_f / len_f
        aux = NT / (NT - 1.0) * ((NT - 1.0) * f_row * g_row
                                 + (1.0 - f_row) * (1.0 - g_row))
        aux_ref[0] = jnp.full((1, LQ), aux, dtype=jnp.float32)

    @pl.when(jnp.logical_not(i * T < len_b))
    def _masked_tail():
        probs_ref[0] = jnp.zeros((TP, LQ), jnp.float32)
        bmask_ref[0] = jnp.zeros((TP, LQ), jnp.int32)


def _tc_probs(tokens, lens, w_q, w_k, skt):
    return pl.pallas_call(
        _tc_body,
        grid=(B, NBL),
        in_specs=[
            pl.BlockSpec(memory_space=pltpu.SMEM),
            pl.BlockSpec((1, T, D), lambda b, i: (b, i, 0)),
            pl.BlockSpec((D, D), lambda b, i: (0, 0)),
            pl.BlockSpec((D, D), lambda b, i: (0, 0)),
            pl.BlockSpec((1, D), lambda b, i: (0, 0)),
        ],
        out_specs=[
            pl.BlockSpec((1, TP, LQ), lambda b, i: (b, i, 0)),
            pl.BlockSpec((1, TP, LQ), lambda b, i: (b, i, 0)),
            pl.BlockSpec((1, 1, LQ), lambda b, i: (b, 0, 0)),
        ],
        out_shape=[
            jax.ShapeDtypeStruct((B, LP, LQ), jnp.float32),
            jax.ShapeDtypeStruct((B, LP, LQ), jnp.int32),
            jax.ShapeDtypeStruct((B, 1, LQ), jnp.float32),
        ],
        scratch_shapes=[
            pltpu.VMEM((1, D), jnp.float32),
            pltpu.SMEM((2,), jnp.float32),
        ],
        compiler_params=pltpu.CompilerParams(
            dimension_semantics=("arbitrary", "arbitrary")),
    )(lens, tokens, w_q, w_k, skt)


def _sc_body(tokens_ref, probs_ref, bmask_ref, lens_ref,
             ds_ref, gates_ref, cl_ref,
             lens_v, bm_v, pr_v, sel_v, bp_v, cl_v, gt_v,
             rows_v, rows2_v, zeros_v, sem, sem2, wsem, wsem2):
    c = lax.axis_index("c")
    s = lax.axis_index("s")
    wid = c * 16 + s
    row = wid // QUARTERS
    quarter = wid % QUARTERS
    jbase = quarter * JSEG
    base = row * L

    pltpu.sync_copy(lens_ref, lens_v)
    pltpu.sync_copy(bmask_ref.at[row], bm_v)
    pltpu.sync_copy(probs_ref.at[row], pr_v)
    len_r = lens_v[pl.ds(row, 16)][0]

    def _init(g, _):
        z16 = jnp.zeros((16,), jnp.int32)
        sel_v[pl.ds(g * 16, 16)] = z16
        bp_v[pl.ds(g * 16, 16)] = z16.astype(jnp.float32)
        return 0
    lax.fori_loop(0, (L + 16) // 16, _init, 0)

    def _compact(g, off):
        bm = bm_v[pl.ds(g * 16, 16)]
        msk = bm > 0
        posv = g * 16 + lax.iota(jnp.int32, 16)
        csum = plsc.cumsum(bm)
        dest = jnp.where(msk, off + csum - bm, DUMP)
        plsc.store_scatter(sel_v, [dest], base + posv)
        plsc.store_scatter(bp_v, [dest], pr_v[pl.ds(g * 16, 16)])
        return off + csum[15]
    nc = lax.fori_loop(0, L // 16, _compact, 0)
    lane0 = lax.iota(jnp.int32, 16) == 0
    plsc.store_scatter(sel_v, [jnp.where(lane0, nc, DUMP)],
                       jnp.full((16,), base + len_r, jnp.int32))

    @pl.when(quarter == 0)
    def _derive_out():
        def _derive(g, _):
            a = sel_v[pl.ds(g * 16, 16)]
            b2 = sel_v[pl.ds(g * 16 + 1, 16)]
            jv = g * 16 + lax.iota(jnp.int32, 16)
            cl_v[pl.ds(g * 16, 16)] = jnp.where(jv < nc, b2 - a, 0)
            gt_v[pl.ds(g * 16, 16)] = 1.0 - bp_v[pl.ds(g * 16, 16)]
            return 0
        lax.fori_loop(0, L // 16, _derive, 0)
        pltpu.sync_copy(cl_v, cl_ref.at[row])
        pltpu.sync_copy(gt_v, gates_ref.at[row])

    def _zinit(j, _):
        for dd in range(D // 16):
            zeros_v[j, pl.ds(dd * 16, 16)] = jnp.zeros((16,), jnp.float32)
        return 0
    lax.fori_loop(0, CH, _zinit, 0)

    def _j0(k):
        return (quarter + QUARTERS * k) * CH

    def _active(k):
        return jnp.logical_and(k < NCH, _j0(k) < nc)

    def _issue(k, buf, gsem):
        @pl.when(_active(k))
        def _():
            pltpu.async_copy(
                tokens_ref.at[sel_v.at[pl.ds(_j0(k), CH)]], buf, gsem)

    def _drain(k, buf, gsem):
        j0 = _j0(k)
        out_slice = ds_ref.at[pl.ds(base + j0, CH)]

        @pl.when(j0 < nc)
        def _():
            pltpu.make_async_copy(
                tokens_ref.at[sel_v.at[pl.ds(j0, CH)]], buf, gsem).wait()

            def _scale(j, _2):
                scl = bp_v[pl.ds(j0 + j, 16)][0]
                for dd in range(D // 16):
                    v = buf[j, pl.ds(dd * 16, 16)]
                    buf[j, pl.ds(dd * 16, 16)] = v * scl
                return 0
            lax.fori_loop(0, CH, _scale, 0)
            pltpu.sync_copy(buf, out_slice)

        @pl.when(jnp.logical_not(j0 < nc))
        def _():
            pltpu.sync_copy(zeros_v, out_slice)

    _issue(0, rows_v, sem)
    _issue(1, rows2_v, sem2)

    def _pair(i, _):
        g0 = 2 * i
        _drain(g0, rows_v, sem)
        _issue(g0 + 2, rows_v, sem)
        _drain(g0 + 1, rows2_v, sem2)
        _issue(g0 + 3, rows2_v, sem2)
        return 0
    lax.fori_loop(0, NCH // 2, _pair, 0)


@functools.partial(
    pl.kernel,
    out_type=[
        jax.ShapeDtypeStruct((B * L, D), jnp.float32),
        jax.ShapeDtypeStruct((B, L), jnp.float32),
        jax.ShapeDtypeStruct((B, L), jnp.int32),
    ],
    mesh=plsc.VectorSubcoreMesh(core_axis_name="c", subcore_axis_name="s"),
    compiler_params=pltpu.CompilerParams(needs_layout_passes=False),
    scratch_types=[
        pltpu.VMEM((32,), jnp.int32),
        pltpu.VMEM((L,), jnp.int32),
        pltpu.VMEM((L,), jnp.float32),
        pltpu.VMEM((L + 16,), jnp.int32),
        pltpu.VMEM((L + 16,), jnp.float32),
        pltpu.VMEM((L,), jnp.int32),
        pltpu.VMEM((L,), jnp.float32),
        pltpu.VMEM((CH, D), jnp.float32),
        pltpu.VMEM((CH, D), jnp.float32),
        pltpu.VMEM((CH, D), jnp.float32),
        pltpu.SemaphoreType.DMA,
        pltpu.SemaphoreType.DMA,
        pltpu.SemaphoreType.DMA,
        pltpu.SemaphoreType.DMA,
    ],
)
def _sc_compact(tokens_ref, probs_ref, bmask_ref, lens_ref,
                ds_ref, gates_ref, cl_ref, *scratch):
    _sc_body(tokens_ref, probs_ref, bmask_ref, lens_ref,
             ds_ref, gates_ref, cl_ref, *scratch)


def kernel(tokens, lens, W_qk, start_key_token):
    w_q = W_qk[:D]
    w_k = W_qk[D:]
    skt = start_key_token.reshape(1, D)
    probs3, bmask3, aux3 = _tc_probs(tokens, lens, w_q, w_k, skt)
    probs = probs3.reshape(B, L)
    bmask_i = bmask3.reshape(B, L)
    lens_pad = jnp.concatenate([lens, jnp.zeros((32 - B,), jnp.int32)])

    ds2, gates, chunk_lens = _sc_compact(
        tokens.reshape(B * L, D), probs, bmask_i, lens_pad)

    downsampled = ds2.reshape(B, L, D)
    boundary_mask = bmask_i.astype(bool)
    upsampler_output_scale = jnp.ones((B, L), jnp.float32)
    weighted_aux_loss = jnp.mean(aux3[:, 0, 0]) * 0.03
    return (downsampled, gates, chunk_lens, boundary_mask,
            upsampler_output_scale, weighted_aux_loss)
